# baseline (speedup 1.0000x reference)
import functools
import jax
import jax.numpy as jnp
from jax.experimental import pallas as pl
from jax.experimental.pallas import tpu as pltpu

N = 100000
H = 64
B = 8
BLK = 4000
NBLK = N // BLK


def _layer_body(agg_ref, hprev_ref, wl_ref, bl_ref, wr_ref, out_ref):
    agg = agg_ref[...]
    agg = jnp.where(jnp.isfinite(agg), agg, 0.0)
    h = hprev_ref[...]
    acc = jnp.dot(agg, wl_ref[...], preferred_element_type=jnp.float32)
    acc += jnp.dot(h, wr_ref[...], preferred_element_type=jnp.float32)
    out_ref[...] = jnp.maximum(acc + bl_ref[...], 0.0)


def _sage_linear(agg, hprev, Wl, bl, Wr):
    k = agg.shape[1]
    grid = (NBLK,)
    return pl.pallas_call(
        _layer_body,
        grid=grid,
        in_specs=[
            pl.BlockSpec((BLK, k), lambda i: (i, 0)),
            pl.BlockSpec((BLK, k), lambda i: (i, 0)),
            pl.BlockSpec((k, H), lambda i: (0, 0)),
            pl.BlockSpec((1, H), lambda i: (0, 0)),
            pl.BlockSpec((k, H), lambda i: (0, 0)),
        ],
        out_specs=pl.BlockSpec((BLK, H), lambda i: (i, 0)),
        out_shape=jax.ShapeDtypeStruct((N, H), jnp.float32),
    )(agg, hprev, Wl, bl.reshape(1, H), Wr)


def _finalize_body(agg3_ref, h2_ref, costs_ref, batch_ref, budget_ref,
                   wl3_ref, bl3_ref, wr3_ref,
                   e1_ref, eb1_ref, e2_ref, eb2_ref,
                   g1_ref, gb1_ref, g2_ref, gb2_ref,
                   a1_ref, ab1_ref, a2_ref, ab2_ref,
                   logits_ref, alpha_ref, gmax_ref):
    i = pl.program_id(0)
    agg3 = agg3_ref[...]
    agg3 = jnp.where(jnp.isfinite(agg3), agg3, 0.0)
    h2 = h2_ref[...]
    h_topo = jnp.dot(agg3, wl3_ref[...], preferred_element_type=jnp.float32)
    h_topo += jnp.dot(h2, wr3_ref[...], preferred_element_type=jnp.float32)
    h_topo = jnp.maximum(h_topo + bl3_ref[...], 0.0)

    batch = batch_ref[0]
    budget = budget_ref[...]
    bsel = jnp.zeros((BLK, 1), jnp.float32)
    for b in range(B):
        bsel = jnp.where(batch == b, budget[0, b], bsel)
    costs = costs_ref[...]
    e1 = e1_ref[...]
    eco1 = costs * e1[0:1, :] + bsel * e1[1:2, :]
    eco1 = jnp.maximum(eco1 + eb1_ref[...], 0.0)
    h_eco = jnp.dot(eco1, e2_ref[...], preferred_element_type=jnp.float32)
    h_eco = jnp.maximum(h_eco + eb2_ref[...], 0.0)

    g1 = g1_ref[...]
    gpre = jnp.dot(h_topo, g1[:H], preferred_element_type=jnp.float32)
    gpre += jnp.dot(h_eco, g1[H:], preferred_element_type=jnp.float32)
    t = jnp.tanh(gpre + gb1_ref[...])
    apre = jnp.dot(t, g2_ref[...], preferred_element_type=jnp.float32) + gb2_ref[...]
    alpha = jax.nn.sigmoid(apre)
    h_final = alpha * h_topo + (1.0 - alpha) * h_eco

    l1 = jnp.maximum(
        jnp.dot(h_final, a1_ref[...], preferred_element_type=jnp.float32)
        + ab1_ref[...], 0.0)
    logits_ref[...] = (
        jnp.dot(l1, a2_ref[...], preferred_element_type=jnp.float32) + ab2_ref[...])
    alpha_ref[...] = alpha

    @pl.when(i == 0)
    def _():
        gmax_ref[...] = jnp.full((B, H), -jnp.inf, jnp.float32)

    parts = jnp.concatenate([
        jnp.max(jnp.where(batch == b, h_final, -jnp.inf), axis=0, keepdims=True)
        for b in range(B)
    ], axis=0)
    gmax_ref[...] = jnp.maximum(gmax_ref[...], parts)


def _head_body(gmax_ref, c1_ref, cb1_ref, c2_ref, cb2_ref, out_ref):
    g = gmax_ref[...]
    g = jnp.where(jnp.isfinite(g), g, 0.0)
    v = jnp.maximum(
        jnp.dot(g, c1_ref[...], preferred_element_type=jnp.float32) + cb1_ref[...],
        0.0)
    out_ref[...] = (
        jnp.dot(v, c2_ref[...], preferred_element_type=jnp.float32) + cb2_ref[...])


def _finalize(agg3, h2, costs, batch3d, budget_row,
              W_l3, b_l3, W_r3, E1, eb1, E2, eb2, G1, gb1, G2, gb2,
              A1, ab1, A2, ab2):
    grid = (NBLK,)

    def bspec(shape):
        nz = len(shape)
        return pl.BlockSpec(shape, lambda i, _n=nz: tuple(0 for _ in range(_n)))

    outs = pl.pallas_call(
        _finalize_body,
        grid=grid,
        in_specs=[
            pl.BlockSpec((BLK, H), lambda i: (i, 0)),
            pl.BlockSpec((BLK, H), lambda i: (i, 0)),
            pl.BlockSpec((BLK, 1), lambda i: (i, 0)),
            pl.BlockSpec((1, BLK, 1), lambda i: (i, 0, 0)),
            bspec((1, B)),
            bspec((H, H)), bspec((1, H)), bspec((H, H)),
            bspec((2, H)), bspec((1, H)), bspec((H, H)), bspec((1, H)),
            bspec((2 * H, 32)), bspec((1, 32)), bspec((32, 1)), bspec((1, 1)),
            bspec((H, H)), bspec((1, H)), bspec((H, 1)), bspec((1, 1)),
        ],
        out_specs=[
            pl.BlockSpec((BLK, 1), lambda i: (i, 0)),
            pl.BlockSpec((BLK, 1), lambda i: (i, 0)),
            pl.BlockSpec((B, H), lambda i: (0, 0)),
        ],
        out_shape=[
            jax.ShapeDtypeStruct((N, 1), jnp.float32),
            jax.ShapeDtypeStruct((N, 1), jnp.float32),
            jax.ShapeDtypeStruct((B, H), jnp.float32),
        ],
    )(agg3, h2, costs, batch3d, budget_row,
      W_l3, b_l3.reshape(1, H), W_r3,
      E1, eb1.reshape(1, H), E2, eb2.reshape(1, H),
      G1, gb1.reshape(1, 32), G2, gb2.reshape(1, 1),
      A1, ab1.reshape(1, H), A2, ab2.reshape(1, 1))
    return outs


def _head(gmax, C1, cb1, C2, cb2):
    return pl.pallas_call(
        _head_body,
        out_shape=jax.ShapeDtypeStruct((B, 1), jnp.float32),
    )(gmax, C1, cb1.reshape(1, H), C2, cb2.reshape(1, 1))


@jax.jit
def kernel(x, edge_index, costs, budget, batch, W_l1, b_l1, W_r1, W_l2, b_l2,
           W_r2, W_l3, b_l3, W_r3, E1, eb1, E2, eb2, G1, gb1, G2, gb2,
           A1, ab1, A2, ab2, C1, cb1, C2, cb2):
    src, dst = edge_index[0], edge_index[1]

    def segmax(vals):
        return jax.ops.segment_max(vals[src], dst, num_segments=N)

    x_pad = jnp.pad(x, ((0, 0), (0, 6)))
    Wl1_pad = jnp.pad(W_l1, ((0, 6), (0, 0)))
    Wr1_pad = jnp.pad(W_r1, ((0, 6), (0, 0)))
    agg1 = segmax(x_pad)
    h1 = _sage_linear(agg1, x_pad, Wl1_pad, b_l1, Wr1_pad)
    agg2 = segmax(h1)
    h2 = _sage_linear(agg2, h1, W_l2, b_l2, W_r2)
    agg3 = segmax(h2)

    batch3d = batch.reshape(NBLK, BLK, 1)
    budget_row = budget.reshape(1, B)
    node_logits, alpha, gmax = _finalize(
        agg3, h2, costs, batch3d, budget_row,
        W_l3, b_l3, W_r3, E1, eb1, E2, eb2, G1, gb1, G2, gb2,
        A1, ab1, A2, ab2)
    value = _head(gmax, C1, cb1, C2, cb2)
    return (node_logits, value, alpha)


# speedup vs baseline: 4.5450x; 4.5450x over previous
import functools
import jax
import jax.numpy as jnp
from jax import lax
from jax.experimental import pallas as pl
from jax.experimental.pallas import tpu as pltpu
from jax.experimental.pallas import tpu_sc as plsc

N = 100000
E = 1600000
H = 64
B = 8
BLK = 4000
NBLK = N // BLK

NC = 2
NS = 16
NR = 32
RANGE = N // NR
CAP = 57344
LCAP = CAP // 16
CWIN = 2560
NWIN = E // CWIN
SUPER = 4096
NSUP = CAP // SUPER
CHUNK = 128
NCH = SUPER // CHUNK
AGGROWS = 3136

def _compact_body(src_hbm, dst_hbm, srcs_out, dstl_out, sbuf, lbuf, wsrc, wdst,
                  curb, sem):
    c = lax.axis_index("c")
    s = lax.axis_index("s")
    rid = 2 * s + c
    lo = rid * RANGE
    iota = lax.iota(jnp.int32, 16)

    def initb(i, _):
        sbuf[pl.ds(i * 16, 16)] = lo + lax.rem(i * 16 + iota, RANGE)
        lbuf[pl.ds(i * 16, 16)] = jnp.full((16,), RANGE, jnp.int32)
        return 0

    lax.fori_loop(0, CAP // 16, initb, 0, unroll=4)

    curb[pl.ds(0, 16)] = iota * LCAP
    lim = iota * LCAP + LCAP

    pltpu.async_copy(src_hbm.at[pl.ds(0, CWIN)], wsrc.at[0], sem)
    pltpu.async_copy(dst_hbm.at[pl.ds(0, CWIN)], wdst.at[0], sem)

    def win_body(w, _):
        b = lax.rem(w, 2)
        nb = 1 - b
        pltpu.make_async_copy(src_hbm.at[pl.ds(w * CWIN, CWIN)], wsrc.at[b], sem).wait()
        pltpu.make_async_copy(dst_hbm.at[pl.ds(w * CWIN, CWIN)], wdst.at[b], sem).wait()

        @pl.when(w + 1 < NWIN)
        def _():
            pltpu.async_copy(src_hbm.at[pl.ds((w + 1) * CWIN, CWIN)], wsrc.at[nb], sem)
            pltpu.async_copy(dst_hbm.at[pl.ds((w + 1) * CWIN, CWIN)], wdst.at[nb], sem)

        def vbody(i, _):
            sv = wsrc[b, pl.ds(i * 16, 16)]
            dv = wdst[b, pl.ds(i * 16, 16)]
            dl = dv - lo
            m = (dl >= 0) & (dl < RANGE)
            cur = curb[pl.ds(0, 16)]
            okm = m & (cur < lim)
            plsc.store_scatter(sbuf, [cur], sv, mask=okm)
            plsc.store_scatter(lbuf, [cur], dl, mask=okm)
            curb[pl.ds(0, 16)] = cur + okm.astype(jnp.int32)
            return 0

        lax.fori_loop(0, CWIN // 16, vbody, 0)
        return 0

    lax.fori_loop(0, NWIN, win_body, 0)
    pltpu.sync_copy(sbuf, srcs_out.at[rid])
    pltpu.sync_copy(lbuf, dstl_out.at[rid])


def _agg_one_range(table, srcs_hbm, dstl_hbm, agg, sbufs, lbufs, rows,
                   sem_l, sem_g0, sem_g1, rid, F):
    ninf = jnp.full((16,), -jnp.inf, jnp.float32)

    def initb(i, _):
        agg[pl.ds(i * 16, 16)] = ninf
        return 0

    lax.fori_loop(0, AGGROWS * F // 16, initb, 0, unroll=8)

    pltpu.sync_copy(srcs_hbm.at[rid, pl.ds(0, SUPER)], sbufs.at[0])
    pltpu.sync_copy(dstl_hbm.at[rid, pl.ds(0, SUPER)], lbufs.at[0])

    sems = (sem_g0, sem_g1)

    def super_body(ss, _):
        b = lax.rem(ss, 2)
        nb = 1 - b

        @pl.when(ss + 1 < NSUP)
        def _():
            pltpu.async_copy(srcs_hbm.at[rid, pl.ds((ss + 1) * SUPER, SUPER)],
                             sbufs.at[nb], sem_l)
            pltpu.async_copy(dstl_hbm.at[rid, pl.ds((ss + 1) * SUPER, SUPER)],
                             lbufs.at[nb], sem_l)

        def fire(j, p):
            pltpu.async_copy(table.at[sbufs.at[b, pl.ds(j * CHUNK, CHUNK)]],
                             rows.at[p], sems[p])

        def drain(j, p):
            pltpu.make_async_copy(
                table.at[sbufs.at[b, pl.ds(j * CHUNK, CHUNK)]],
                rows.at[p], sems[p]).wait()

        fire(0, 0)

        def kbody(k, _):
            par = lax.rem(k, 2)

            @pl.when((k + 1 < NCH) & (par == 0))
            def _():
                fire(k + 1, 1)

            @pl.when((k + 1 < NCH) & (par == 1))
            def _():
                fire(k + 1, 0)

            @pl.when(par == 0)
            def _():
                drain(k, 0)

            @pl.when(par == 1)
            def _():
                drain(k, 1)

            base = k * CHUNK

            def gbody(g, _):
                dvec = lbufs[b, pl.ds(base + g * 16, 16)] * F
                for e16 in range(16):
                    e = g * 16 + e16
                    ad = dvec[e16]
                    for hh in range(F // 16):
                        a = agg[pl.ds(ad + 16 * hh, 16)]
                        r = rows[par, e, pl.ds(16 * hh, 16)]
                        agg[pl.ds(ad + 16 * hh, 16)] = jnp.maximum(a, r)
                return 0

            lax.fori_loop(0, CHUNK // 16, gbody, 0)
            return 0

        lax.fori_loop(0, NCH, kbody, 0)

        @pl.when(ss + 1 < NSUP)
        def _():
            pltpu.make_async_copy(srcs_hbm.at[rid, pl.ds((ss + 1) * SUPER, SUPER)],
                                  sbufs.at[nb], sem_l).wait()
            pltpu.make_async_copy(dstl_hbm.at[rid, pl.ds((ss + 1) * SUPER, SUPER)],
                                  lbufs.at[nb], sem_l).wait()
        return 0

    lax.fori_loop(0, NSUP, super_body, 0)


def _agg16_body(x16_hbm, srcs_hbm, dstl_hbm, out_hbm, agg, sbufs, lbufs, rows,
                sem_l, sem_g0, sem_g1):
    c = lax.axis_index("c")
    s = lax.axis_index("s")
    rid = 2 * s + c
    _agg_one_range(x16_hbm, srcs_hbm, dstl_hbm, agg, sbufs, lbufs, rows,
                   sem_l, sem_g0, sem_g1, rid, 16)
    pltpu.sync_copy(agg.at[pl.ds(0, RANGE * 16)], out_hbm.at[rid])


def _agg32_body(h3_hbm, srcs_hbm, dstl_hbm, out_hbm, agg, sbufs, lbufs, rows,
                sem_l, sem_g0, sem_g1):
    c = lax.axis_index("c")
    s = lax.axis_index("s")
    table = h3_hbm.at[c]
    for r in range(2):
        rid = 2 * s + r
        _agg_one_range(table, srcs_hbm, dstl_hbm, agg, sbufs, lbufs, rows,
                       sem_l, sem_g0, sem_g1, rid, 32)
        pltpu.sync_copy(agg.at[pl.ds(0, RANGE * 32)], out_hbm.at[c, rid])


@functools.cache
def _sc_kernels():
    mesh = plsc.VectorSubcoreMesh(core_axis_name="c", subcore_axis_name="s")
    cp = pltpu.CompilerParams(use_tc_tiling_on_sc=False, needs_layout_passes=False)
    compact = pl.kernel(
        _compact_body,
        out_type=(
            jax.ShapeDtypeStruct((NR, CAP), jnp.int32),
            jax.ShapeDtypeStruct((NR, CAP), jnp.int32),
        ),
        mesh=mesh,
        compiler_params=cp,
        scratch_types=[
            pltpu.VMEM((CAP,), jnp.int32),
            pltpu.VMEM((CAP,), jnp.int32),
            pltpu.VMEM((2, CWIN), jnp.int32),
            pltpu.VMEM((2, CWIN), jnp.int32),
            pltpu.VMEM((16,), jnp.int32),
            pltpu.SemaphoreType.DMA,
        ],
    )
    agg16 = pl.kernel(
        _agg16_body,
        out_type=jax.ShapeDtypeStruct((NR, RANGE * 16), jnp.float32),
        mesh=mesh,
        compiler_params=cp,
        scratch_types=[
            pltpu.VMEM((AGGROWS * 16,), jnp.float32),
            pltpu.VMEM((2, SUPER), jnp.int32),
            pltpu.VMEM((2, SUPER), jnp.int32),
            pltpu.VMEM((2, CHUNK, 16), jnp.float32),
            pltpu.SemaphoreType.DMA,
            pltpu.SemaphoreType.DMA,
            pltpu.SemaphoreType.DMA,
        ],
    )
    agg32 = pl.kernel(
        _agg32_body,
        out_type=jax.ShapeDtypeStruct((NC, NR, RANGE * 32), jnp.float32),
        mesh=mesh,
        compiler_params=cp,
        scratch_types=[
            pltpu.VMEM((AGGROWS * 32,), jnp.float32),
            pltpu.VMEM((2, SUPER), jnp.int32),
            pltpu.VMEM((2, SUPER), jnp.int32),
            pltpu.VMEM((2, CHUNK, 32), jnp.float32),
            pltpu.SemaphoreType.DMA,
            pltpu.SemaphoreType.DMA,
            pltpu.SemaphoreType.DMA,
        ],
    )
    return compact, agg16, agg32


def _layer_body(agg_ref, hprev_ref, wl_ref, bl_ref, wr_ref, out_ref):
    agg = agg_ref[...]
    agg = jnp.where(jnp.isfinite(agg), agg, 0.0)
    h = hprev_ref[...]
    acc = jnp.dot(agg, wl_ref[...], preferred_element_type=jnp.float32)
    acc += jnp.dot(h, wr_ref[...], preferred_element_type=jnp.float32)
    v = jnp.maximum(acc + bl_ref[...], 0.0)
    out_ref[0] = v[:, :32]
    out_ref[1] = v[:, 32:]


def _sage_linear(agg, hprev, Wl, bl, Wr):
    k = agg.shape[1]
    kp = hprev.shape[1]
    return pl.pallas_call(
        _layer_body,
        grid=(NBLK,),
        in_specs=[
            pl.BlockSpec((BLK, k), lambda i: (i, 0)),
            pl.BlockSpec((BLK, kp), lambda i: (i, 0)),
            pl.BlockSpec((k, H), lambda i: (0, 0)),
            pl.BlockSpec((1, H), lambda i: (0, 0)),
            pl.BlockSpec((kp, H), lambda i: (0, 0)),
        ],
        out_specs=pl.BlockSpec((2, BLK, 32), lambda i: (0, i, 0)),
        out_shape=jax.ShapeDtypeStruct((2, N, 32), jnp.float32),
    )(agg, hprev, Wl, bl.reshape(1, H), Wr)


def _finalize_body(agg3l_ref, agg3h_ref, h2_ref, costs_ref, batch_ref,
                   budget_ref,
                   wl3_ref, bl3_ref, wr3_ref,
                   e1_ref, eb1_ref, e2_ref, eb2_ref,
                   g1_ref, gb1_ref, g2_ref, gb2_ref,
                   a1_ref, ab1_ref, a2_ref, ab2_ref,
                   logits_ref, alpha_ref, gmax_ref):
    i = pl.program_id(0)
    agg3 = jnp.concatenate([agg3l_ref[...], agg3h_ref[...]], axis=1)
    agg3 = jnp.where(jnp.isfinite(agg3), agg3, 0.0)
    h2 = jnp.concatenate([h2_ref[0], h2_ref[1]], axis=1)
    h_topo = jnp.dot(agg3, wl3_ref[...], preferred_element_type=jnp.float32)
    h_topo += jnp.dot(h2, wr3_ref[...], preferred_element_type=jnp.float32)
    h_topo = jnp.maximum(h_topo + bl3_ref[...], 0.0)

    batch = batch_ref[0]
    budget = budget_ref[...]
    bsel = jnp.zeros((BLK, 1), jnp.float32)
    for b in range(B):
        bsel = jnp.where(batch == b, budget[0, b], bsel)
    costs = costs_ref[...]
    e1 = e1_ref[...]
    eco1 = costs * e1[0:1, :] + bsel * e1[1:2, :]
    eco1 = jnp.maximum(eco1 + eb1_ref[...], 0.0)
    h_eco = jnp.dot(eco1, e2_ref[...], preferred_element_type=jnp.float32)
    h_eco = jnp.maximum(h_eco + eb2_ref[...], 0.0)

    g1 = g1_ref[...]
    gpre = jnp.dot(h_topo, g1[:H], preferred_element_type=jnp.float32)
    gpre += jnp.dot(h_eco, g1[H:], preferred_element_type=jnp.float32)
    t = jnp.tanh(gpre + gb1_ref[...])
    apre = jnp.dot(t, g2_ref[...], preferred_element_type=jnp.float32) + gb2_ref[...]
    alpha = jax.nn.sigmoid(apre)
    h_final = alpha * h_topo + (1.0 - alpha) * h_eco

    l1 = jnp.maximum(
        jnp.dot(h_final, a1_ref[...], preferred_element_type=jnp.float32)
        + ab1_ref[...], 0.0)
    logits_ref[...] = (
        jnp.dot(l1, a2_ref[...], preferred_element_type=jnp.float32) + ab2_ref[...])
    alpha_ref[...] = alpha

    @pl.when(i == 0)
    def _():
        gmax_ref[...] = jnp.full((B, H), -jnp.inf, jnp.float32)

    parts = jnp.concatenate([
        jnp.max(jnp.where(batch == b, h_final, -jnp.inf), axis=0, keepdims=True)
        for b in range(B)
    ], axis=0)
    gmax_ref[...] = jnp.maximum(gmax_ref[...], parts)


def _head_body(gmax_ref, c1_ref, cb1_ref, c2_ref, cb2_ref, out_ref):
    g = gmax_ref[...]
    g = jnp.where(jnp.isfinite(g), g, 0.0)
    v = jnp.maximum(
        jnp.dot(g, c1_ref[...], preferred_element_type=jnp.float32) + cb1_ref[...],
        0.0)
    out_ref[...] = (
        jnp.dot(v, c2_ref[...], preferred_element_type=jnp.float32) + cb2_ref[...])


def _finalize(agg3l, agg3h, h2_3, costs, batch3d, budget_row,
              W_l3, b_l3, W_r3, E1, eb1, E2, eb2, G1, gb1, G2, gb2,
              A1, ab1, A2, ab2):
    def bspec(shape):
        nz = len(shape)
        return pl.BlockSpec(shape, lambda i, _n=nz: tuple(0 for _ in range(_n)))

    return pl.pallas_call(
        _finalize_body,
        grid=(NBLK,),
        in_specs=[
            pl.BlockSpec((BLK, 32), lambda i: (i, 0)),
            pl.BlockSpec((BLK, 32), lambda i: (i, 0)),
            pl.BlockSpec((2, BLK, 32), lambda i: (0, i, 0)),
            pl.BlockSpec((BLK, 1), lambda i: (i, 0)),
            pl.BlockSpec((1, BLK, 1), lambda i: (i, 0, 0)),
            bspec((1, B)),
            bspec((H, H)), bspec((1, H)), bspec((H, H)),
            bspec((2, H)), bspec((1, H)), bspec((H, H)), bspec((1, H)),
            bspec((2 * H, 32)), bspec((1, 32)), bspec((32, 1)), bspec((1, 1)),
            bspec((H, H)), bspec((1, H)), bspec((H, 1)), bspec((1, 1)),
        ],
        out_specs=[
            pl.BlockSpec((BLK, 1), lambda i: (i, 0)),
            pl.BlockSpec((BLK, 1), lambda i: (i, 0)),
            pl.BlockSpec((B, H), lambda i: (0, 0)),
        ],
        out_shape=[
            jax.ShapeDtypeStruct((N, 1), jnp.float32),
            jax.ShapeDtypeStruct((N, 1), jnp.float32),
            jax.ShapeDtypeStruct((B, H), jnp.float32),
        ],
    )(agg3l, agg3h, h2_3, costs, batch3d, budget_row,
      W_l3, b_l3.reshape(1, H), W_r3,
      E1, eb1.reshape(1, H), E2, eb2.reshape(1, H),
      G1, gb1.reshape(1, 32), G2, gb2.reshape(1, 1),
      A1, ab1.reshape(1, H), A2, ab2.reshape(1, 1))


def _head(gmax, C1, cb1, C2, cb2):
    return pl.pallas_call(
        _head_body,
        out_shape=jax.ShapeDtypeStruct((B, 1), jnp.float32),
    )(gmax, C1, cb1.reshape(1, H), C2, cb2.reshape(1, 1))


@jax.jit
def kernel(x, edge_index, costs, budget, batch, W_l1, b_l1, W_r1, W_l2, b_l2,
           W_r2, W_l3, b_l3, W_r3, E1, eb1, E2, eb2, G1, gb1, G2, gb2,
           A1, ab1, A2, ab2, C1, cb1, C2, cb2):
    src = edge_index[0]
    dst = edge_index[1]

    _compact, _agg16, _agg32 = _sc_kernels()
    srcs, dstls = _compact(src, dst)

    x16 = jnp.pad(x, ((0, 0), (0, 14)))
    agg1 = _agg16(x16, srcs, dstls).reshape(N, 16)
    Wl1_pad = jnp.pad(W_l1, ((0, 14), (0, 0)))
    Wr1_pad = jnp.pad(W_r1, ((0, 14), (0, 0)))
    h1_3 = _sage_linear(agg1, x16, Wl1_pad, b_l1, Wr1_pad)

    a2 = _agg32(h1_3, srcs, dstls)
    agg2l = a2[0].reshape(N, 32)
    agg2h = a2[1].reshape(N, 32)
    agg2 = jnp.concatenate([agg2l, agg2h], axis=1)
    h1 = jnp.concatenate([h1_3[0], h1_3[1]], axis=1)
    h2_3 = _sage_linear(agg2, h1, W_l2, b_l2, W_r2)

    a3 = _agg32(h2_3, srcs, dstls)
    agg3l = a3[0].reshape(N, 32)
    agg3h = a3[1].reshape(N, 32)

    batch3d = batch.reshape(NBLK, BLK, 1)
    budget_row = budget.reshape(1, B)
    node_logits, alpha, gmax = _finalize(
        agg3l, agg3h, h2_3, costs, batch3d, budget_row,
        W_l3, b_l3, W_r3, E1, eb1, E2, eb2, G1, gb1, G2, gb2,
        A1, ab1, A2, ab2)
    value = _head(gmax, C1, cb1, C2, cb2)
    return (node_logits, value, alpha)


# speedup vs baseline: 4.8300x; 1.0627x over previous
import functools
import jax
import jax.numpy as jnp
from jax import lax
from jax.experimental import pallas as pl
from jax.experimental.pallas import tpu as pltpu
from jax.experimental.pallas import tpu_sc as plsc

N = 100000
E = 1600000
H = 64
B = 8
BLK = 4000
NBLK = N // BLK

NC = 2
NS = 16
NR = 32
RANGE = N // NR
CAP = 57344
LCAP = CAP // 16
CWIN = 2560
NWIN = E // CWIN
SUPER = 4096
NSUP = CAP // SUPER
CHUNK = 128
NCH = SUPER // CHUNK
AGGROWS = 3136

def _compact_body(edge_hbm, srcs_out, dstl_out, sbuf, lbuf, wsrc, wdst,
                  curb, sem):
    src_hbm = edge_hbm.at[0]
    dst_hbm = edge_hbm.at[1]
    c = lax.axis_index("c")
    s = lax.axis_index("s")
    rid = 2 * s + c
    lo = rid * RANGE
    iota = lax.iota(jnp.int32, 16)

    def initb(i, _):
        sbuf[pl.ds(i * 16, 16)] = lo + lax.rem(i * 16 + iota, RANGE)
        lbuf[pl.ds(i * 16, 16)] = jnp.full((16,), RANGE, jnp.int32)
        return 0

    lax.fori_loop(0, CAP // 16, initb, 0, unroll=4)

    curb[pl.ds(0, 16)] = iota * LCAP
    lim = iota * LCAP + LCAP

    pltpu.async_copy(src_hbm.at[pl.ds(0, CWIN)], wsrc.at[0], sem)
    pltpu.async_copy(dst_hbm.at[pl.ds(0, CWIN)], wdst.at[0], sem)

    def win_body(w, _):
        b = lax.rem(w, 2)
        nb = 1 - b
        pltpu.make_async_copy(src_hbm.at[pl.ds(w * CWIN, CWIN)], wsrc.at[b], sem).wait()
        pltpu.make_async_copy(dst_hbm.at[pl.ds(w * CWIN, CWIN)], wdst.at[b], sem).wait()

        @pl.when(w + 1 < NWIN)
        def _():
            pltpu.async_copy(src_hbm.at[pl.ds((w + 1) * CWIN, CWIN)], wsrc.at[nb], sem)
            pltpu.async_copy(dst_hbm.at[pl.ds((w + 1) * CWIN, CWIN)], wdst.at[nb], sem)

        def vbody(i, _):
            sv = wsrc[b, pl.ds(i * 16, 16)]
            dv = wdst[b, pl.ds(i * 16, 16)]
            dl = dv - lo
            m = (dl >= 0) & (dl < RANGE)
            cur = curb[pl.ds(0, 16)]
            okm = m & (cur < lim)
            plsc.store_scatter(sbuf, [cur], sv, mask=okm)
            plsc.store_scatter(lbuf, [cur], dl, mask=okm)
            curb[pl.ds(0, 16)] = cur + okm.astype(jnp.int32)
            return 0

        lax.fori_loop(0, CWIN // 16, vbody, 0, unroll=2)
        return 0

    lax.fori_loop(0, NWIN, win_body, 0)
    pltpu.sync_copy(sbuf, srcs_out.at[rid])
    pltpu.sync_copy(lbuf, dstl_out.at[rid])


def _agg_one_range(table, srcs_hbm, dstl_hbm, agg, sbufs, lbufs, rows,
                   sem_l, sem_g0, sem_g1, rid, F):
    ninf = jnp.full((16,), -jnp.inf, jnp.float32)

    def initb(i, _):
        agg[pl.ds(i * 16, 16)] = ninf
        return 0

    lax.fori_loop(0, AGGROWS * F // 16, initb, 0, unroll=8)

    pltpu.sync_copy(srcs_hbm.at[rid, pl.ds(0, SUPER)], sbufs.at[0])
    pltpu.sync_copy(dstl_hbm.at[rid, pl.ds(0, SUPER)], lbufs.at[0])

    sems = (sem_g0, sem_g1)

    def super_body(ss, _):
        b = lax.rem(ss, 2)
        nb = 1 - b

        @pl.when(ss + 1 < NSUP)
        def _():
            pltpu.async_copy(srcs_hbm.at[rid, pl.ds((ss + 1) * SUPER, SUPER)],
                             sbufs.at[nb], sem_l)
            pltpu.async_copy(dstl_hbm.at[rid, pl.ds((ss + 1) * SUPER, SUPER)],
                             lbufs.at[nb], sem_l)

        def fire(j, p):
            pltpu.async_copy(table.at[sbufs.at[b, pl.ds(j * CHUNK, CHUNK)]],
                             rows.at[p], sems[p])

        def drain(j, p):
            pltpu.make_async_copy(
                table.at[sbufs.at[b, pl.ds(j * CHUNK, CHUNK)]],
                rows.at[p], sems[p]).wait()

        fire(0, 0)

        def kbody(k, _):
            par = lax.rem(k, 2)

            @pl.when((k + 1 < NCH) & (par == 0))
            def _():
                fire(k + 1, 1)

            @pl.when((k + 1 < NCH) & (par == 1))
            def _():
                fire(k + 1, 0)

            @pl.when(par == 0)
            def _():
                drain(k, 0)

            @pl.when(par == 1)
            def _():
                drain(k, 1)

            base = k * CHUNK

            def gbody(g, _):
                dvec = lbufs[b, pl.ds(base + g * 16, 16)] * F
                for e16 in range(16):
                    e = g * 16 + e16
                    ad = dvec[e16]
                    for hh in range(F // 16):
                        a = agg[pl.ds(ad + 16 * hh, 16)]
                        r = rows[par, e, pl.ds(16 * hh, 16)]
                        agg[pl.ds(ad + 16 * hh, 16)] = jnp.maximum(a, r)
                return 0

            lax.fori_loop(0, CHUNK // 16, gbody, 0, unroll=2)
            return 0

        lax.fori_loop(0, NCH, kbody, 0)

        @pl.when(ss + 1 < NSUP)
        def _():
            pltpu.make_async_copy(srcs_hbm.at[rid, pl.ds((ss + 1) * SUPER, SUPER)],
                                  sbufs.at[nb], sem_l).wait()
            pltpu.make_async_copy(dstl_hbm.at[rid, pl.ds((ss + 1) * SUPER, SUPER)],
                                  lbufs.at[nb], sem_l).wait()
        return 0

    lax.fori_loop(0, NSUP, super_body, 0)


def _agg16_body(x16_hbm, srcs_hbm, dstl_hbm, out_hbm, agg, sbufs, lbufs, rows,
                sem_l, sem_g0, sem_g1):
    c = lax.axis_index("c")
    s = lax.axis_index("s")
    rid = 2 * s + c
    _agg_one_range(x16_hbm, srcs_hbm, dstl_hbm, agg, sbufs, lbufs, rows,
                   sem_l, sem_g0, sem_g1, rid, 16)
    pltpu.sync_copy(agg.at[pl.ds(0, RANGE * 16)], out_hbm.at[rid])


def _agg32_body(h3_hbm, srcs_hbm, dstl_hbm, out_hbm, agg, sbufs, lbufs, rows,
                sem_l, sem_g0, sem_g1):
    c = lax.axis_index("c")
    s = lax.axis_index("s")
    table = h3_hbm.at[c]
    for r in range(2):
        rid = 2 * s + r
        _agg_one_range(table, srcs_hbm, dstl_hbm, agg, sbufs, lbufs, rows,
                       sem_l, sem_g0, sem_g1, rid, 32)
        pltpu.sync_copy(agg.at[pl.ds(0, RANGE * 32)], out_hbm.at[c, rid])


@functools.cache
def _sc_kernels():
    mesh = plsc.VectorSubcoreMesh(core_axis_name="c", subcore_axis_name="s")
    cp = pltpu.CompilerParams(use_tc_tiling_on_sc=False, needs_layout_passes=False)
    compact = pl.kernel(
        _compact_body,
        out_type=(
            jax.ShapeDtypeStruct((NR, CAP), jnp.int32),
            jax.ShapeDtypeStruct((NR, CAP), jnp.int32),
        ),
        mesh=mesh,
        compiler_params=cp,
        scratch_types=[
            pltpu.VMEM((CAP,), jnp.int32),
            pltpu.VMEM((CAP,), jnp.int32),
            pltpu.VMEM((2, CWIN), jnp.int32),
            pltpu.VMEM((2, CWIN), jnp.int32),
            pltpu.VMEM((16,), jnp.int32),
            pltpu.SemaphoreType.DMA,
        ],
    )
    agg16 = pl.kernel(
        _agg16_body,
        out_type=jax.ShapeDtypeStruct((NR, RANGE * 16), jnp.float32),
        mesh=mesh,
        compiler_params=cp,
        scratch_types=[
            pltpu.VMEM((AGGROWS * 16,), jnp.float32),
            pltpu.VMEM((2, SUPER), jnp.int32),
            pltpu.VMEM((2, SUPER), jnp.int32),
            pltpu.VMEM((2, CHUNK, 16), jnp.float32),
            pltpu.SemaphoreType.DMA,
            pltpu.SemaphoreType.DMA,
            pltpu.SemaphoreType.DMA,
        ],
    )
    agg32 = pl.kernel(
        _agg32_body,
        out_type=jax.ShapeDtypeStruct((NC, NR, RANGE * 32), jnp.float32),
        mesh=mesh,
        compiler_params=cp,
        scratch_types=[
            pltpu.VMEM((AGGROWS * 32,), jnp.float32),
            pltpu.VMEM((2, SUPER), jnp.int32),
            pltpu.VMEM((2, SUPER), jnp.int32),
            pltpu.VMEM((2, CHUNK, 32), jnp.float32),
            pltpu.SemaphoreType.DMA,
            pltpu.SemaphoreType.DMA,
            pltpu.SemaphoreType.DMA,
        ],
    )
    return compact, agg16, agg32


def _layer_body(agg_ref, hprev_ref, wl_ref, bl_ref, wr_ref, out_ref):
    agg = agg_ref[...]
    agg = jnp.where(jnp.isfinite(agg), agg, 0.0)
    h = hprev_ref[...]
    acc = jnp.dot(agg, wl_ref[...], preferred_element_type=jnp.float32)
    acc += jnp.dot(h, wr_ref[...], preferred_element_type=jnp.float32)
    v = jnp.maximum(acc + bl_ref[...], 0.0)
    out_ref[0] = v[:, :32]
    out_ref[1] = v[:, 32:]


def _sage_linear(agg, hprev, Wl, bl, Wr):
    k = agg.shape[1]
    kp = hprev.shape[1]
    return pl.pallas_call(
        _layer_body,
        grid=(NBLK,),
        in_specs=[
            pl.BlockSpec((BLK, k), lambda i: (i, 0)),
            pl.BlockSpec((BLK, kp), lambda i: (i, 0)),
            pl.BlockSpec((k, H), lambda i: (0, 0)),
            pl.BlockSpec((1, H), lambda i: (0, 0)),
            pl.BlockSpec((kp, H), lambda i: (0, 0)),
        ],
        out_specs=pl.BlockSpec((2, BLK, 32), lambda i: (0, i, 0)),
        out_shape=jax.ShapeDtypeStruct((2, N, 32), jnp.float32),
    )(agg, hprev, Wl, bl.reshape(1, H), Wr)


def _layer_body2(aggl_ref, aggh_ref, h3_ref, wl_ref, bl_ref, wr_ref, out_ref):
    agg = jnp.concatenate([aggl_ref[...], aggh_ref[...]], axis=1)
    agg = jnp.where(jnp.isfinite(agg), agg, 0.0)
    h = jnp.concatenate([h3_ref[0], h3_ref[1]], axis=1)
    acc = jnp.dot(agg, wl_ref[...], preferred_element_type=jnp.float32)
    acc += jnp.dot(h, wr_ref[...], preferred_element_type=jnp.float32)
    v = jnp.maximum(acc + bl_ref[...], 0.0)
    out_ref[0] = v[:, :32]
    out_ref[1] = v[:, 32:]


def _sage_linear2(aggl, aggh, h3prev, Wl, bl, Wr):
    return pl.pallas_call(
        _layer_body2,
        grid=(NBLK,),
        in_specs=[
            pl.BlockSpec((BLK, 32), lambda i: (i, 0)),
            pl.BlockSpec((BLK, 32), lambda i: (i, 0)),
            pl.BlockSpec((2, BLK, 32), lambda i: (0, i, 0)),
            pl.BlockSpec((H, H), lambda i: (0, 0)),
            pl.BlockSpec((1, H), lambda i: (0, 0)),
            pl.BlockSpec((H, H), lambda i: (0, 0)),
        ],
        out_specs=pl.BlockSpec((2, BLK, 32), lambda i: (0, i, 0)),
        out_shape=jax.ShapeDtypeStruct((2, N, 32), jnp.float32),
    )(aggl, aggh, h3prev, Wl, bl.reshape(1, H), Wr)


def _finalize_body(agg3l_ref, agg3h_ref, h2_ref, costs_ref, batch_ref,
                   budget_ref,
                   wl3_ref, bl3_ref, wr3_ref,
                   e1_ref, eb1_ref, e2_ref, eb2_ref,
                   g1_ref, gb1_ref, g2_ref, gb2_ref,
                   a1_ref, ab1_ref, a2_ref, ab2_ref,
                   logits_ref, alpha_ref, gmax_ref):
    i = pl.program_id(0)
    agg3 = jnp.concatenate([agg3l_ref[...], agg3h_ref[...]], axis=1)
    agg3 = jnp.where(jnp.isfinite(agg3), agg3, 0.0)
    h2 = jnp.concatenate([h2_ref[0], h2_ref[1]], axis=1)
    h_topo = jnp.dot(agg3, wl3_ref[...], preferred_element_type=jnp.float32)
    h_topo += jnp.dot(h2, wr3_ref[...], preferred_element_type=jnp.float32)
    h_topo = jnp.maximum(h_topo + bl3_ref[...], 0.0)

    batch = batch_ref[0]
    budget = budget_ref[...]
    bsel = jnp.zeros((BLK, 1), jnp.float32)
    for b in range(B):
        bsel = jnp.where(batch == b, budget[0, b], bsel)
    costs = costs_ref[...]
    e1 = e1_ref[...]
    eco1 = costs * e1[0:1, :] + bsel * e1[1:2, :]
    eco1 = jnp.maximum(eco1 + eb1_ref[...], 0.0)
    h_eco = jnp.dot(eco1, e2_ref[...], preferred_element_type=jnp.float32)
    h_eco = jnp.maximum(h_eco + eb2_ref[...], 0.0)

    g1 = g1_ref[...]
    gpre = jnp.dot(h_topo, g1[:H], preferred_element_type=jnp.float32)
    gpre += jnp.dot(h_eco, g1[H:], preferred_element_type=jnp.float32)
    t = jnp.tanh(gpre + gb1_ref[...])
    apre = jnp.dot(t, g2_ref[...], preferred_element_type=jnp.float32) + gb2_ref[...]
    alpha = jax.nn.sigmoid(apre)
    h_final = alpha * h_topo + (1.0 - alpha) * h_eco

    l1 = jnp.maximum(
        jnp.dot(h_final, a1_ref[...], preferred_element_type=jnp.float32)
        + ab1_ref[...], 0.0)
    logits_ref[...] = (
        jnp.dot(l1, a2_ref[...], preferred_element_type=jnp.float32) + ab2_ref[...])
    alpha_ref[...] = alpha

    @pl.when(i == 0)
    def _():
        gmax_ref[...] = jnp.full((B, H), -jnp.inf, jnp.float32)

    parts = jnp.concatenate([
        jnp.max(jnp.where(batch == b, h_final, -jnp.inf), axis=0, keepdims=True)
        for b in range(B)
    ], axis=0)
    gmax_ref[...] = jnp.maximum(gmax_ref[...], parts)


def _head_body(gmax_ref, c1_ref, cb1_ref, c2_ref, cb2_ref, out_ref):
    g = gmax_ref[...]
    g = jnp.where(jnp.isfinite(g), g, 0.0)
    v = jnp.maximum(
        jnp.dot(g, c1_ref[...], preferred_element_type=jnp.float32) + cb1_ref[...],
        0.0)
    out_ref[...] = (
        jnp.dot(v, c2_ref[...], preferred_element_type=jnp.float32) + cb2_ref[...])


def _finalize(agg3l, agg3h, h2_3, costs, batch3d, budget_row,
              W_l3, b_l3, W_r3, E1, eb1, E2, eb2, G1, gb1, G2, gb2,
              A1, ab1, A2, ab2):
    def bspec(shape):
        nz = len(shape)
        return pl.BlockSpec(shape, lambda i, _n=nz: tuple(0 for _ in range(_n)))

    return pl.pallas_call(
        _finalize_body,
        grid=(NBLK,),
        in_specs=[
            pl.BlockSpec((BLK, 32), lambda i: (i, 0)),
            pl.BlockSpec((BLK, 32), lambda i: (i, 0)),
            pl.BlockSpec((2, BLK, 32), lambda i: (0, i, 0)),
            pl.BlockSpec((BLK, 1), lambda i: (i, 0)),
            pl.BlockSpec((1, BLK, 1), lambda i: (i, 0, 0)),
            bspec((1, B)),
            bspec((H, H)), bspec((1, H)), bspec((H, H)),
            bspec((2, H)), bspec((1, H)), bspec((H, H)), bspec((1, H)),
            bspec((2 * H, 32)), bspec((1, 32)), bspec((32, 1)), bspec((1, 1)),
            bspec((H, H)), bspec((1, H)), bspec((H, 1)), bspec((1, 1)),
        ],
        out_specs=[
            pl.BlockSpec((BLK, 1), lambda i: (i, 0)),
            pl.BlockSpec((BLK, 1), lambda i: (i, 0)),
            pl.BlockSpec((B, H), lambda i: (0, 0)),
        ],
        out_shape=[
            jax.ShapeDtypeStruct((N, 1), jnp.float32),
            jax.ShapeDtypeStruct((N, 1), jnp.float32),
            jax.ShapeDtypeStruct((B, H), jnp.float32),
        ],
    )(agg3l, agg3h, h2_3, costs, batch3d, budget_row,
      W_l3, b_l3.reshape(1, H), W_r3,
      E1, eb1.reshape(1, H), E2, eb2.reshape(1, H),
      G1, gb1.reshape(1, 32), G2, gb2.reshape(1, 1),
      A1, ab1.reshape(1, H), A2, ab2.reshape(1, 1))


def _head(gmax, C1, cb1, C2, cb2):
    return pl.pallas_call(
        _head_body,
        out_shape=jax.ShapeDtypeStruct((B, 1), jnp.float32),
    )(gmax, C1, cb1.reshape(1, H), C2, cb2.reshape(1, 1))


@jax.jit
def kernel(x, edge_index, costs, budget, batch, W_l1, b_l1, W_r1, W_l2, b_l2,
           W_r2, W_l3, b_l3, W_r3, E1, eb1, E2, eb2, G1, gb1, G2, gb2,
           A1, ab1, A2, ab2, C1, cb1, C2, cb2):
    _compact, _agg16, _agg32 = _sc_kernels()
    srcs, dstls = _compact(edge_index)

    x16 = jnp.pad(x, ((0, 0), (0, 14)))
    agg1 = _agg16(x16, srcs, dstls).reshape(N, 16)
    Wl1_pad = jnp.pad(W_l1, ((0, 14), (0, 0)))
    Wr1_pad = jnp.pad(W_r1, ((0, 14), (0, 0)))
    h1_3 = _sage_linear(agg1, x16, Wl1_pad, b_l1, Wr1_pad)

    a2 = _agg32(h1_3, srcs, dstls)
    h2_3 = _sage_linear2(a2[0].reshape(N, 32), a2[1].reshape(N, 32), h1_3,
                         W_l2, b_l2, W_r2)

    a3 = _agg32(h2_3, srcs, dstls)
    agg3l = a3[0].reshape(N, 32)
    agg3h = a3[1].reshape(N, 32)

    batch3d = batch.reshape(NBLK, BLK, 1)
    budget_row = budget.reshape(1, B)
    node_logits, alpha, gmax = _finalize(
        agg3l, agg3h, h2_3, costs, batch3d, budget_row,
        W_l3, b_l3, W_r3, E1, eb1, E2, eb2, G1, gb1, G2, gb2,
        A1, ab1, A2, ab2)
    value = _head(gmax, C1, cb1, C2, cb2)
    return (node_logits, value, alpha)
